# baseline (speedup 1.0000x reference)
import functools

import jax
import jax.numpy as jnp
from jax.experimental import pallas as pl
from jax.experimental.pallas import tpu as pltpu
from jax.experimental.pallas import tpu_sc as plsc

B, N, C_FEAT = 8, 8192, 64
NPOINT, NSAMPLE = 1024, 32
IN_CH = C_FEAT + 3
ROW = 128
GCHUNK = 256


def _fps_kernel(x_ref, y_ref, z_ref, t_ref, idx_ref, nx_ref, ny_ref, nz_ref):
    x = x_ref[...]
    y = y_ref[...]
    z = z_ref[...]
    tbl = t_ref[...]
    lane = jax.lax.broadcasted_iota(jnp.int32, (B, N), 1)
    lane128 = jax.lax.broadcasted_iota(jnp.int32, (B, 128), 1)
    sub128 = jax.lax.broadcasted_iota(jnp.int32, (B, 128), 0)
    mx = (lane128 == 3 * sub128).astype(jnp.float32)
    my = (lane128 == 3 * sub128 + 1).astype(jnp.float32)
    mz = (lane128 == 3 * sub128 + 2).astype(jnp.float32)
    px, py, pz = x[:, 0:1], y[:, 0:1], z[:, 0:1]
    dists = jnp.full((B, N), jnp.inf, jnp.float32)

    CHUNK = 128

    def body(i, carry):
        dists, px, py, pz, bi, bx, by, bz = carry
        dx = x - px
        dy = y - py
        dz = z - pz
        d = dx * dx + dy * dy + dz * dz
        dists = jnp.minimum(dists, d)
        m = jnp.max(dists, axis=1, keepdims=True)
        sel = jnp.min(jnp.where(dists == m, lane, N), axis=1, keepdims=True)
        sel = jnp.where(i == 0, 0, sel)
        oh = (lane == sel).astype(jnp.float32)
        p = jnp.dot(oh, tbl, preferred_element_type=jnp.float32,
                    precision=jax.lax.Precision.HIGHEST)
        px = jnp.sum(p * mx, axis=1, keepdims=True)
        py = jnp.sum(p * my, axis=1, keepdims=True)
        pz = jnp.sum(p * mz, axis=1, keepdims=True)
        j = i % CHUNK
        hit = lane128 == j
        bi = jnp.where(hit, sel, bi)
        bx = jnp.where(hit, px, bx)
        by = jnp.where(hit, py, by)
        bz = jnp.where(hit, pz, bz)
        return dists, px, py, pz, bi, bx, by, bz

    bi = jnp.zeros((B, CHUNK), jnp.int32)
    bx = jnp.zeros((B, CHUNK), jnp.float32)
    by = jnp.zeros((B, CHUNK), jnp.float32)
    bz = jnp.zeros((B, CHUNK), jnp.float32)
    carry = (dists, px, py, pz, bi, bx, by, bz)
    for c in range(NPOINT // CHUNK):
        carry = jax.lax.fori_loop(c * CHUNK, (c + 1) * CHUNK, body, carry)
        _, _, _, _, bi, bx, by, bz = carry
        sl = slice(c * CHUNK, (c + 1) * CHUNK)
        idx_ref[:, sl] = bi
        nx_ref[:, sl] = bx
        ny_ref[:, sl] = by
        nz_ref[:, sl] = bz


def _run_fps(xyz):
    xt = jnp.transpose(xyz, (0, 2, 1))
    x, y, z = xt[:, 0], xt[:, 1], xt[:, 2]
    tbl = jnp.pad(jnp.transpose(xyz, (1, 0, 2)).reshape(N, 3 * B),
                  ((0, 0), (0, 128 - 3 * B)))
    idx, nx, ny, nz = pl.pallas_call(
        _fps_kernel,
        out_shape=(
            jax.ShapeDtypeStruct((B, NPOINT), jnp.int32),
            jax.ShapeDtypeStruct((B, NPOINT), jnp.float32),
            jax.ShapeDtypeStruct((B, NPOINT), jnp.float32),
            jax.ShapeDtypeStruct((B, NPOINT), jnp.float32),
        ),
    )(x, y, z, tbl)
    return idx, jnp.stack([nx, ny, nz], axis=-1)


QB = 128


def _topk_kernel(c8_ref, q8_ref, gi_ref):
    c8 = c8_ref[0]
    q8 = q8_ref[0]
    cn2 = jnp.sum(c8 * c8, axis=1, keepdims=True)
    qn2 = jnp.sum(q8 * q8, axis=1)[None, :]
    d = cn2 + qn2 - 2.0 * jnp.dot(c8, q8.T, preferred_element_type=jnp.float32)
    d3 = d.reshape(NSAMPLE, N // NSAMPLE, QB)
    vals = [d3[a] for a in range(NSAMPLE)]
    base = jax.lax.broadcasted_iota(jnp.int32, (N // NSAMPLE, QB), 0)
    idxs = [base + a * (N // NSAMPLE) for a in range(NSAMPLE)]

    def ce(i, j, asc):
        ad, ax, bd, bx = vals[i], idxs[i], vals[j], idxs[j]
        sw = bd < ad
        lo_d = jnp.where(sw, bd, ad)
        lo_x = jnp.where(sw, bx, ax)
        hi_d = jnp.where(sw, ad, bd)
        hi_x = jnp.where(sw, ax, bx)
        if asc:
            vals[i], idxs[i], vals[j], idxs[j] = lo_d, lo_x, hi_d, hi_x
        else:
            vals[i], idxs[i], vals[j], idxs[j] = hi_d, hi_x, lo_d, lo_x

    k = 2
    while k <= NSAMPLE:
        j = k // 2
        while j >= 1:
            for i in range(NSAMPLE):
                p = i ^ j
                if p > i:
                    ce(i, p, asc=((i & k) == 0))
            j //= 2
        k *= 2

    R = N // NSAMPLE
    while R > 1:
        h = R // 2
        nv, nx = [], []
        for i in range(NSAMPLE):
            ad, ax = vals[i][:h], idxs[i][:h]
            bd, bx = vals[NSAMPLE - 1 - i][h:], idxs[NSAMPLE - 1 - i][h:]
            sw = bd < ad
            nv.append(jnp.where(sw, bd, ad))
            nx.append(jnp.where(sw, bx, ax))
        vals, idxs = nv, nx
        j = NSAMPLE // 2
        while j >= 1:
            for i in range(NSAMPLE):
                p = i ^ j
                if p > i:
                    ce(i, p, asc=True)
            j //= 2
        R = h

    gi = jnp.concatenate(idxs, axis=0)
    gi_ref[0, 0] = gi.T


def _run_topk(xyz, new_xyz):
    c8 = jnp.pad(xyz, ((0, 0), (0, 0), (0, 5)))
    q8 = jnp.pad(new_xyz, ((0, 0), (0, 0), (0, 5)))
    nqb = NPOINT // QB
    gi = pl.pallas_call(
        _topk_kernel,
        grid=(B, nqb),
        in_specs=[
            pl.BlockSpec((1, N, 8), lambda b, q: (b, 0, 0)),
            pl.BlockSpec((1, QB, 8), lambda b, q: (b, q, 0)),
        ],
        out_specs=pl.BlockSpec((1, 1, QB, NSAMPLE), lambda b, q: (b, q, 0, 0)),
        out_shape=jax.ShapeDtypeStruct((B, nqb, QB, NSAMPLE), jnp.int32),
    )(c8, q8)
    return gi.reshape(B, NPOINT, NSAMPLE)


def _sc_gather(table, idx):
    info = plsc.get_sparse_core_info()
    nw = info.num_cores * info.num_subcores
    rows_total = idx.shape[0]
    b_per_w = rows_total // nw
    nchunk = b_per_w // GCHUNK
    mesh = plsc.VectorSubcoreMesh(core_axis_name="c", subcore_axis_name="s")

    @functools.partial(
        pl.kernel,
        mesh=mesh,
        out_type=jax.ShapeDtypeStruct((rows_total, ROW), jnp.float32),
        scratch_types=[
            pltpu.VMEM((GCHUNK,), jnp.int32),
            pltpu.VMEM((GCHUNK,), jnp.int32),
            pltpu.VMEM((GCHUNK, ROW), jnp.float32),
            pltpu.VMEM((GCHUNK, ROW), jnp.float32),
            pltpu.SemaphoreType.DMA,
            pltpu.SemaphoreType.DMA,
        ],
    )
    def gather_k(table_hbm, idx_hbm, out_hbm, idx_a, idx_b, rows_a, rows_b,
                 sem_a, sem_b):
        wid = jax.lax.axis_index("s") * info.num_cores + jax.lax.axis_index("c")
        base = wid * b_per_w
        idxs = (idx_a, idx_b)
        bufs = (rows_a, rows_b)
        sems = (sem_a, sem_b)
        copies = [None, None]
        for c in range(nchunk):
            s = c % 2
            if copies[s] is not None:
                copies[s].wait()
                pltpu.sync_copy(bufs[s],
                                out_hbm.at[pl.ds(base + (c - 2) * GCHUNK, GCHUNK)])
            pltpu.sync_copy(idx_hbm.at[pl.ds(base + c * GCHUNK, GCHUNK)], idxs[s])
            copies[s] = pltpu.async_copy(table_hbm.at[idxs[s]], bufs[s], sems[s])
        for c in range(nchunk - 2, nchunk):
            s = c % 2
            copies[s].wait()
            pltpu.sync_copy(bufs[s],
                            out_hbm.at[pl.ds(base + c * GCHUNK, GCHUNK)])

    return gather_k(table, idx)


def _mlp_kernel(rows_ref, q8_ref, w80_ref, wx8_ref, a1_ref, w2_ref, a2_ref,
                w3_ref, a3_ref, out_ref):
    g = rows_ref[0]
    q8 = q8_ref[0]

    def bn_relu(h, a_ref):
        m = jnp.mean(h, axis=0, keepdims=True)
        v = jnp.mean((h - m) ** 2, axis=0, keepdims=True)
        xh = (h - m) / jnp.sqrt(v + 1e-5)
        return jnp.maximum(xh * a_ref[0:1] + a_ref[1:2], 0.0)

    h = jnp.dot(g, w80_ref[...].T, preferred_element_type=jnp.float32)
    corr = jnp.dot(q8, wx8_ref[...].T, preferred_element_type=jnp.float32)
    h = (h.reshape(NPOINT, NSAMPLE, -1) - corr[:, None, :])
    h = h.reshape(NPOINT * NSAMPLE, -1) + a1_ref[2:3]
    h = bn_relu(h, a1_ref)
    h = jnp.dot(h, w2_ref[...].T, preferred_element_type=jnp.float32) + a2_ref[2:3]
    h = bn_relu(h, a2_ref)
    h = jnp.dot(h, w3_ref[...].T, preferred_element_type=jnp.float32) + a3_ref[2:3]
    h = bn_relu(h, a3_ref)
    h = h.reshape(NPOINT, NSAMPLE, h.shape[-1])
    out_ref[0] = jnp.max(h, axis=1)


def _run_mlp(rows, q8, params):
    (w80, wx8, a1), (w2, a2), (w3, a3) = params
    cout = w3.shape[0]
    specs = [
        pl.BlockSpec((1, NPOINT * NSAMPLE, ROW), lambda b: (b, 0, 0)),
        pl.BlockSpec((1, NPOINT, 8), lambda b: (b, 0, 0)),
        pl.BlockSpec(w80.shape, lambda b: (0, 0)),
        pl.BlockSpec(wx8.shape, lambda b: (0, 0)),
        pl.BlockSpec(a1.shape, lambda b: (0, 0)),
        pl.BlockSpec(w2.shape, lambda b: (0, 0)),
        pl.BlockSpec(a2.shape, lambda b: (0, 0)),
        pl.BlockSpec(w3.shape, lambda b: (0, 0)),
        pl.BlockSpec(a3.shape, lambda b: (0, 0)),
    ]
    return pl.pallas_call(
        _mlp_kernel,
        grid=(B,),
        in_specs=specs,
        out_specs=pl.BlockSpec((1, NPOINT, cout), lambda b: (b, 0, 0)),
        out_shape=jax.ShapeDtypeStruct((B, NPOINT, cout), jnp.float32),
    )(rows, q8, w80, wx8, a1, w2, a2, w3, a3)


def kernel(xyz, features, W1, b1, g1, beta1, W2, b2, g2, beta2, W3, b3, g3, beta3):
    fps_idx, new_xyz = _run_fps(xyz)
    group_idx = _run_topk(xyz, new_xyz)

    table = jnp.concatenate(
        [xyz, jnp.transpose(features, (0, 2, 1)),
         jnp.zeros((B, N, ROW - IN_CH), jnp.float32)], axis=-1,
    ).reshape(B * N, ROW)
    flat_idx = (group_idx + (jnp.arange(B, dtype=jnp.int32) * N)[:, None, None])
    rows = _sc_gather(table, flat_idx.reshape(-1))

    q8 = jnp.pad(new_xyz, ((0, 0), (0, 0), (0, 5)))
    w80 = jnp.pad(W1, ((0, 0), (0, ROW - IN_CH)))
    wx8 = jnp.pad(W1[:, :3], ((0, 0), (0, 5)))
    params = [(w80, wx8, jnp.stack([g1, beta1, b1])),
              (W2, jnp.stack([g2, beta2, b2])),
              (W3, jnp.stack([g3, beta3, b3]))]
    out = _run_mlp(rows.reshape(B, NPOINT * NSAMPLE, ROW), q8, params)
    return new_xyz, out


# speedup vs baseline: 1.8423x; 1.8423x over previous
import functools

import jax
import jax.numpy as jnp
from jax.experimental import pallas as pl
from jax.experimental.pallas import tpu as pltpu
from jax.experimental.pallas import tpu_sc as plsc

B, N, C_FEAT = 8, 8192, 64
NPOINT, NSAMPLE = 1024, 32
IN_CH = C_FEAT + 3
ROW = 128
GCHUNK = 256


def _fps_kernel(x_ref, y_ref, z_ref, idx_ref, nx_ref, ny_ref, nz_ref):
    x = x_ref[...]
    y = y_ref[...]
    z = z_ref[...]
    lane = jax.lax.broadcasted_iota(jnp.int32, (B, N), 1)
    lane128 = jax.lax.broadcasted_iota(jnp.int32, (B, 128), 1)
    px, py, pz = x[:, 0:1], y[:, 0:1], z[:, 0:1]
    dists = jnp.full((B, N), jnp.inf, jnp.float32)

    CHUNK = 128

    def body(i, carry):
        dists, px, py, pz, bi, bx, by, bz = carry
        dx = x - px
        dy = y - py
        dz = z - pz
        d = dx * dx + dy * dy + dz * dz
        dists = jnp.minimum(dists, d)
        m = jnp.max(dists, axis=1, keepdims=True)
        sel = jnp.min(jnp.where(dists == m, lane, N), axis=1, keepdims=True)
        sel = jnp.where(i == 0, 0, sel)
        oh = lane == sel
        px = jnp.sum(jnp.where(oh, x, 0.0), axis=1, keepdims=True)
        py = jnp.sum(jnp.where(oh, y, 0.0), axis=1, keepdims=True)
        pz = jnp.sum(jnp.where(oh, z, 0.0), axis=1, keepdims=True)
        j = i % CHUNK
        hit = lane128 == j
        bi = jnp.where(hit, sel, bi)
        bx = jnp.where(hit, px, bx)
        by = jnp.where(hit, py, by)
        bz = jnp.where(hit, pz, bz)
        return dists, px, py, pz, bi, bx, by, bz

    bi = jnp.zeros((B, CHUNK), jnp.int32)
    bx = jnp.zeros((B, CHUNK), jnp.float32)
    by = jnp.zeros((B, CHUNK), jnp.float32)
    bz = jnp.zeros((B, CHUNK), jnp.float32)
    carry = (dists, px, py, pz, bi, bx, by, bz)
    for c in range(NPOINT // CHUNK):
        carry = jax.lax.fori_loop(c * CHUNK, (c + 1) * CHUNK, body, carry)
        _, _, _, _, bi, bx, by, bz = carry
        sl = slice(c * CHUNK, (c + 1) * CHUNK)
        idx_ref[:, sl] = bi
        nx_ref[:, sl] = bx
        ny_ref[:, sl] = by
        nz_ref[:, sl] = bz


def _run_fps(xyz):
    xt = jnp.transpose(xyz, (0, 2, 1))
    x, y, z = xt[:, 0], xt[:, 1], xt[:, 2]
    idx, nx, ny, nz = pl.pallas_call(
        _fps_kernel,
        out_shape=(
            jax.ShapeDtypeStruct((B, NPOINT), jnp.int32),
            jax.ShapeDtypeStruct((B, NPOINT), jnp.float32),
            jax.ShapeDtypeStruct((B, NPOINT), jnp.float32),
            jax.ShapeDtypeStruct((B, NPOINT), jnp.float32),
        ),
    )(x, y, z)
    return idx, jnp.stack([nx, ny, nz], axis=-1)


QB = 128


def _topk_kernel(c8_ref, q8_ref, gi_ref):
    c8 = c8_ref[0]
    q8 = q8_ref[0]
    cn2 = jnp.sum(c8 * c8, axis=1, keepdims=True)
    qn2 = jnp.sum(q8 * q8, axis=1)[None, :]
    d = cn2 + qn2 - 2.0 * jnp.dot(c8, q8.T, preferred_element_type=jnp.float32)
    d3 = d.reshape(NSAMPLE, N // NSAMPLE, QB)
    vals = [d3[a] for a in range(NSAMPLE)]
    base = jax.lax.broadcasted_iota(jnp.int32, (N // NSAMPLE, QB), 0)
    idxs = [base + a * (N // NSAMPLE) for a in range(NSAMPLE)]

    def ce(i, j, asc):
        ad, ax, bd, bx = vals[i], idxs[i], vals[j], idxs[j]
        sw = bd < ad
        lo_d = jnp.where(sw, bd, ad)
        lo_x = jnp.where(sw, bx, ax)
        hi_d = jnp.where(sw, ad, bd)
        hi_x = jnp.where(sw, ax, bx)
        if asc:
            vals[i], idxs[i], vals[j], idxs[j] = lo_d, lo_x, hi_d, hi_x
        else:
            vals[i], idxs[i], vals[j], idxs[j] = hi_d, hi_x, lo_d, lo_x

    k = 2
    while k <= NSAMPLE:
        j = k // 2
        while j >= 1:
            for i in range(NSAMPLE):
                p = i ^ j
                if p > i:
                    ce(i, p, asc=((i & k) == 0))
            j //= 2
        k *= 2

    R = N // NSAMPLE
    while R > 1:
        h = R // 2
        nv, nx = [], []
        for i in range(NSAMPLE):
            ad, ax = vals[i][:h], idxs[i][:h]
            bd, bx = vals[NSAMPLE - 1 - i][h:], idxs[NSAMPLE - 1 - i][h:]
            sw = bd < ad
            nv.append(jnp.where(sw, bd, ad))
            nx.append(jnp.where(sw, bx, ax))
        vals, idxs = nv, nx
        j = NSAMPLE // 2
        while j >= 1:
            for i in range(NSAMPLE):
                p = i ^ j
                if p > i:
                    ce(i, p, asc=True)
            j //= 2
        R = h

    gi = jnp.concatenate(idxs, axis=0)
    gi_ref[0, 0] = gi.T


def _run_topk(xyz, new_xyz):
    c8 = jnp.pad(xyz, ((0, 0), (0, 0), (0, 5)))
    q8 = jnp.pad(new_xyz, ((0, 0), (0, 0), (0, 5)))
    nqb = NPOINT // QB
    gi = pl.pallas_call(
        _topk_kernel,
        grid=(B, nqb),
        in_specs=[
            pl.BlockSpec((1, N, 8), lambda b, q: (b, 0, 0)),
            pl.BlockSpec((1, QB, 8), lambda b, q: (b, q, 0)),
        ],
        out_specs=pl.BlockSpec((1, 1, QB, NSAMPLE), lambda b, q: (b, q, 0, 0)),
        out_shape=jax.ShapeDtypeStruct((B, nqb, QB, NSAMPLE), jnp.int32),
    )(c8, q8)
    return gi.reshape(B, NPOINT, NSAMPLE)


def _sc_gather(table, idx):
    info = plsc.get_sparse_core_info()
    nw = info.num_cores * info.num_subcores
    rows_total = idx.shape[0]
    b_per_w = rows_total // nw
    nchunk = b_per_w // GCHUNK
    mesh = plsc.VectorSubcoreMesh(core_axis_name="c", subcore_axis_name="s")

    @functools.partial(
        pl.kernel,
        mesh=mesh,
        out_type=jax.ShapeDtypeStruct((rows_total, ROW), jnp.float32),
        scratch_types=[
            pltpu.VMEM((GCHUNK,), jnp.int32),
            pltpu.VMEM((GCHUNK,), jnp.int32),
            pltpu.VMEM((GCHUNK, ROW), jnp.float32),
            pltpu.VMEM((GCHUNK, ROW), jnp.float32),
            pltpu.SemaphoreType.DMA,
            pltpu.SemaphoreType.DMA,
        ],
    )
    def gather_k(table_hbm, idx_hbm, out_hbm, idx_a, idx_b, rows_a, rows_b,
                 sem_a, sem_b):
        wid = jax.lax.axis_index("s") * info.num_cores + jax.lax.axis_index("c")
        base = wid * b_per_w
        idxs = (idx_a, idx_b)
        bufs = (rows_a, rows_b)
        sems = (sem_a, sem_b)
        copies = [None, None]
        for c in range(nchunk):
            s = c % 2
            if copies[s] is not None:
                copies[s].wait()
                pltpu.sync_copy(bufs[s],
                                out_hbm.at[pl.ds(base + (c - 2) * GCHUNK, GCHUNK)])
            pltpu.sync_copy(idx_hbm.at[pl.ds(base + c * GCHUNK, GCHUNK)], idxs[s])
            copies[s] = pltpu.async_copy(table_hbm.at[idxs[s]], bufs[s], sems[s])
        for c in range(nchunk - 2, nchunk):
            s = c % 2
            copies[s].wait()
            pltpu.sync_copy(bufs[s],
                            out_hbm.at[pl.ds(base + c * GCHUNK, GCHUNK)])

    return gather_k(table, idx)


def _mlp_kernel(rows_ref, q8_ref, w80_ref, wx8_ref, a1_ref, w2_ref, a2_ref,
                w3_ref, a3_ref, out_ref):
    g = rows_ref[0]
    q8 = q8_ref[0]

    def bn_relu(h, a_ref):
        m = jnp.mean(h, axis=0, keepdims=True)
        v = jnp.mean((h - m) ** 2, axis=0, keepdims=True)
        xh = (h - m) / jnp.sqrt(v + 1e-5)
        return jnp.maximum(xh * a_ref[0:1] + a_ref[1:2], 0.0)

    h = jnp.dot(g, w80_ref[...].T, preferred_element_type=jnp.float32)
    corr = jnp.dot(q8, wx8_ref[...].T, preferred_element_type=jnp.float32)
    h = (h.reshape(NPOINT, NSAMPLE, -1) - corr[:, None, :])
    h = h.reshape(NPOINT * NSAMPLE, -1) + a1_ref[2:3]
    h = bn_relu(h, a1_ref)
    h = jnp.dot(h, w2_ref[...].T, preferred_element_type=jnp.float32) + a2_ref[2:3]
    h = bn_relu(h, a2_ref)
    h = jnp.dot(h, w3_ref[...].T, preferred_element_type=jnp.float32) + a3_ref[2:3]
    h = bn_relu(h, a3_ref)
    h = h.reshape(NPOINT, NSAMPLE, h.shape[-1])
    out_ref[0] = jnp.max(h, axis=1)


def _run_mlp(rows, q8, params):
    (w80, wx8, a1), (w2, a2), (w3, a3) = params
    cout = w3.shape[0]
    nb = rows.shape[0]
    specs = [
        pl.BlockSpec((1, NPOINT * NSAMPLE, ROW), lambda b: (b, 0, 0)),
        pl.BlockSpec((1, NPOINT, 8), lambda b: (b, 0, 0)),
        pl.BlockSpec(w80.shape, lambda b: (0, 0)),
        pl.BlockSpec(wx8.shape, lambda b: (0, 0)),
        pl.BlockSpec(a1.shape, lambda b: (0, 0)),
        pl.BlockSpec(w2.shape, lambda b: (0, 0)),
        pl.BlockSpec(a2.shape, lambda b: (0, 0)),
        pl.BlockSpec(w3.shape, lambda b: (0, 0)),
        pl.BlockSpec(a3.shape, lambda b: (0, 0)),
    ]
    return pl.pallas_call(
        _mlp_kernel,
        grid=(nb,),
        in_specs=specs,
        out_specs=pl.BlockSpec((1, NPOINT, cout), lambda b: (b, 0, 0)),
        out_shape=jax.ShapeDtypeStruct((nb, NPOINT, cout), jnp.float32),
    )(rows, q8, w80, wx8, a1, w2, a2, w3, a3)


def kernel(xyz, features, W1, b1, g1, beta1, W2, b2, g2, beta2, W3, b3, g3, beta3):
    fps_idx, new_xyz = _run_fps(xyz)
    group_idx = _run_topk(xyz, new_xyz)

    table = jnp.concatenate(
        [xyz, jnp.transpose(features, (0, 2, 1)),
         jnp.zeros((B, N, ROW - IN_CH), jnp.float32)], axis=-1,
    ).reshape(B * N, ROW)
    flat_idx = (group_idx + (jnp.arange(B, dtype=jnp.int32) * N)[:, None, None])

    q8 = jnp.pad(new_xyz, ((0, 0), (0, 0), (0, 5)))
    w80 = jnp.pad(W1, ((0, 0), (0, ROW - IN_CH)))
    wx8 = jnp.pad(W1[:, :3], ((0, 0), (0, 5)))
    params = [(w80, wx8, jnp.stack([g1, beta1, b1])),
              (W2, jnp.stack([g2, beta2, b2])),
              (W3, jnp.stack([g3, beta3, b3]))]

    H = B // 2
    outs = []
    for lo in (0, H):
        rows_h = _sc_gather(table, flat_idx[lo:lo + H].reshape(-1))
        outs.append(_run_mlp(rows_h.reshape(H, NPOINT * NSAMPLE, ROW),
                             q8[lo:lo + H], params))
    return new_xyz, jnp.concatenate(outs, axis=0)


# speedup vs baseline: 1.9394x; 1.0527x over previous
import functools

import jax
import jax.numpy as jnp
from jax.experimental import pallas as pl
from jax.experimental.pallas import tpu as pltpu
from jax.experimental.pallas import tpu_sc as plsc

B, N, C_FEAT = 8, 8192, 64
NPOINT, NSAMPLE = 1024, 32
IN_CH = C_FEAT + 3
ROW = 128
GCHUNK = 256


def _fps_kernel(x_ref, y_ref, z_ref, idx_ref, nx_ref, ny_ref, nz_ref):
    x = x_ref[...]
    y = y_ref[...]
    z = z_ref[...]
    lane = jax.lax.broadcasted_iota(jnp.int32, (B, N), 1)
    lane128 = jax.lax.broadcasted_iota(jnp.int32, (B, 128), 1)
    px, py, pz = x[:, 0:1], y[:, 0:1], z[:, 0:1]
    dists = jnp.full((B, N), jnp.inf, jnp.float32)

    CHUNK = 128

    def body(i, carry):
        dists, px, py, pz, bi, bx, by, bz = carry
        dx = x - px
        dy = y - py
        dz = z - pz
        d = dx * dx + dy * dy + dz * dz
        dists = jnp.minimum(dists, d)
        m = jnp.max(dists, axis=1, keepdims=True)
        sel = jnp.min(jnp.where(dists == m, lane, N), axis=1, keepdims=True)
        sel = jnp.where(i == 0, 0, sel)
        oh = lane == sel
        px = jnp.sum(jnp.where(oh, x, 0.0), axis=1, keepdims=True)
        py = jnp.sum(jnp.where(oh, y, 0.0), axis=1, keepdims=True)
        pz = jnp.sum(jnp.where(oh, z, 0.0), axis=1, keepdims=True)
        j = i % CHUNK
        hit = lane128 == j
        bi = jnp.where(hit, sel, bi)
        bx = jnp.where(hit, px, bx)
        by = jnp.where(hit, py, by)
        bz = jnp.where(hit, pz, bz)
        return dists, px, py, pz, bi, bx, by, bz

    bi = jnp.zeros((B, CHUNK), jnp.int32)
    bx = jnp.zeros((B, CHUNK), jnp.float32)
    by = jnp.zeros((B, CHUNK), jnp.float32)
    bz = jnp.zeros((B, CHUNK), jnp.float32)
    carry = (dists, px, py, pz, bi, bx, by, bz)
    for c in range(NPOINT // CHUNK):
        carry = jax.lax.fori_loop(c * CHUNK, (c + 1) * CHUNK, body, carry)
        _, _, _, _, bi, bx, by, bz = carry
        sl = slice(c * CHUNK, (c + 1) * CHUNK)
        idx_ref[:, sl] = bi
        nx_ref[:, sl] = bx
        ny_ref[:, sl] = by
        nz_ref[:, sl] = bz


def _run_fps(xyz):
    xt = jnp.transpose(xyz, (0, 2, 1))
    x, y, z = xt[:, 0], xt[:, 1], xt[:, 2]
    idx, nx, ny, nz = pl.pallas_call(
        _fps_kernel,
        out_shape=(
            jax.ShapeDtypeStruct((B, NPOINT), jnp.int32),
            jax.ShapeDtypeStruct((B, NPOINT), jnp.float32),
            jax.ShapeDtypeStruct((B, NPOINT), jnp.float32),
            jax.ShapeDtypeStruct((B, NPOINT), jnp.float32),
        ),
    )(x, y, z)
    return idx, jnp.stack([nx, ny, nz], axis=-1)


QB = 256


def _topk_kernel(c8_ref, q8_ref, gi_ref):
    c8 = c8_ref[0]
    q8 = q8_ref[0]
    cn2 = jnp.sum(c8 * c8, axis=1, keepdims=True)
    qn2 = jnp.sum(q8 * q8, axis=1)[None, :]
    d = cn2 + qn2 - 2.0 * jnp.dot(c8, q8.T, preferred_element_type=jnp.float32)
    d3 = d.reshape(NSAMPLE, N // NSAMPLE, QB)
    vals = [d3[a] for a in range(NSAMPLE)]
    base = jax.lax.broadcasted_iota(jnp.int32, (N // NSAMPLE, QB), 0)
    idxs = [base + a * (N // NSAMPLE) for a in range(NSAMPLE)]

    def ce(i, j, asc):
        ad, ax, bd, bx = vals[i], idxs[i], vals[j], idxs[j]
        sw = bd < ad
        lo_d = jnp.where(sw, bd, ad)
        lo_x = jnp.where(sw, bx, ax)
        hi_d = jnp.where(sw, ad, bd)
        hi_x = jnp.where(sw, ax, bx)
        if asc:
            vals[i], idxs[i], vals[j], idxs[j] = lo_d, lo_x, hi_d, hi_x
        else:
            vals[i], idxs[i], vals[j], idxs[j] = hi_d, hi_x, lo_d, lo_x

    k = 2
    while k <= NSAMPLE:
        j = k // 2
        while j >= 1:
            for i in range(NSAMPLE):
                p = i ^ j
                if p > i:
                    ce(i, p, asc=((i & k) == 0))
            j //= 2
        k *= 2

    R = N // NSAMPLE
    while R > 1:
        h = R // 2
        nv, nx = [], []
        for i in range(NSAMPLE):
            ad, ax = vals[i][:h], idxs[i][:h]
            bd, bx = vals[NSAMPLE - 1 - i][h:], idxs[NSAMPLE - 1 - i][h:]
            sw = bd < ad
            nv.append(jnp.where(sw, bd, ad))
            nx.append(jnp.where(sw, bx, ax))
        vals, idxs = nv, nx
        j = NSAMPLE // 2
        while j >= 1:
            for i in range(NSAMPLE):
                p = i ^ j
                if p > i:
                    ce(i, p, asc=True)
            j //= 2
        R = h

    gi = jnp.concatenate(idxs, axis=0)
    gi_ref[0, 0] = gi.T


def _run_topk(xyz, new_xyz):
    c8 = jnp.pad(xyz, ((0, 0), (0, 0), (0, 5)))
    q8 = jnp.pad(new_xyz, ((0, 0), (0, 0), (0, 5)))
    nqb = NPOINT // QB
    gi = pl.pallas_call(
        _topk_kernel,
        grid=(B, nqb),
        in_specs=[
            pl.BlockSpec((1, N, 8), lambda b, q: (b, 0, 0)),
            pl.BlockSpec((1, QB, 8), lambda b, q: (b, q, 0)),
        ],
        out_specs=pl.BlockSpec((1, 1, QB, NSAMPLE), lambda b, q: (b, q, 0, 0)),
        out_shape=jax.ShapeDtypeStruct((B, nqb, QB, NSAMPLE), jnp.int32),
    )(c8, q8)
    return gi.reshape(B, NPOINT, NSAMPLE)


def _sc_gather(table, idx):
    info = plsc.get_sparse_core_info()
    nw = info.num_cores * info.num_subcores
    rows_total = idx.shape[0]
    b_per_w = rows_total // nw
    nchunk = b_per_w // GCHUNK
    mesh = plsc.VectorSubcoreMesh(core_axis_name="c", subcore_axis_name="s")

    @functools.partial(
        pl.kernel,
        mesh=mesh,
        out_type=jax.ShapeDtypeStruct((rows_total, ROW), jnp.float32),
        scratch_types=[
            pltpu.VMEM((GCHUNK,), jnp.int32),
            pltpu.VMEM((GCHUNK,), jnp.int32),
            pltpu.VMEM((GCHUNK, ROW), jnp.float32),
            pltpu.VMEM((GCHUNK, ROW), jnp.float32),
            pltpu.SemaphoreType.DMA,
            pltpu.SemaphoreType.DMA,
        ],
    )
    def gather_k(table_hbm, idx_hbm, out_hbm, idx_a, idx_b, rows_a, rows_b,
                 sem_a, sem_b):
        wid = jax.lax.axis_index("s") * info.num_cores + jax.lax.axis_index("c")
        base = wid * b_per_w
        idxs = (idx_a, idx_b)
        bufs = (rows_a, rows_b)
        sems = (sem_a, sem_b)
        copies = [None, None]
        for c in range(nchunk):
            s = c % 2
            if copies[s] is not None:
                copies[s].wait()
                pltpu.sync_copy(bufs[s],
                                out_hbm.at[pl.ds(base + (c - 2) * GCHUNK, GCHUNK)])
            pltpu.sync_copy(idx_hbm.at[pl.ds(base + c * GCHUNK, GCHUNK)], idxs[s])
            copies[s] = pltpu.async_copy(table_hbm.at[idxs[s]], bufs[s], sems[s])
        for c in range(nchunk - 2, nchunk):
            s = c % 2
            copies[s].wait()
            pltpu.sync_copy(bufs[s],
                            out_hbm.at[pl.ds(base + c * GCHUNK, GCHUNK)])

    return gather_k(table, idx)


def _mlp_kernel(rows_ref, q8_ref, w80_ref, wx8_ref, a1_ref, w2_ref, a2_ref,
                w3_ref, a3_ref, out_ref):
    g = rows_ref[0]
    q8 = q8_ref[0]

    def bn_relu(h, a_ref):
        m = jnp.mean(h, axis=0, keepdims=True)
        v = jnp.mean((h - m) ** 2, axis=0, keepdims=True)
        xh = (h - m) / jnp.sqrt(v + 1e-5)
        return jnp.maximum(xh * a_ref[0:1] + a_ref[1:2], 0.0)

    h = jnp.dot(g, w80_ref[...].T, preferred_element_type=jnp.float32)
    corr = jnp.dot(q8, wx8_ref[...].T, preferred_element_type=jnp.float32)
    h = (h.reshape(NPOINT, NSAMPLE, -1) - corr[:, None, :])
    h = h.reshape(NPOINT * NSAMPLE, -1) + a1_ref[2:3]
    h = bn_relu(h, a1_ref)
    h = jnp.dot(h, w2_ref[...].T, preferred_element_type=jnp.float32) + a2_ref[2:3]
    h = bn_relu(h, a2_ref)
    h = jnp.dot(h, w3_ref[...].T, preferred_element_type=jnp.float32) + a3_ref[2:3]
    h = bn_relu(h, a3_ref)
    h = h.reshape(NPOINT, NSAMPLE, h.shape[-1])
    out_ref[0] = jnp.max(h, axis=1)


def _run_mlp(rows, q8, params):
    (w80, wx8, a1), (w2, a2), (w3, a3) = params
    cout = w3.shape[0]
    nb = rows.shape[0]
    specs = [
        pl.BlockSpec((1, NPOINT * NSAMPLE, ROW), lambda b: (b, 0, 0)),
        pl.BlockSpec((1, NPOINT, 8), lambda b: (b, 0, 0)),
        pl.BlockSpec(w80.shape, lambda b: (0, 0)),
        pl.BlockSpec(wx8.shape, lambda b: (0, 0)),
        pl.BlockSpec(a1.shape, lambda b: (0, 0)),
        pl.BlockSpec(w2.shape, lambda b: (0, 0)),
        pl.BlockSpec(a2.shape, lambda b: (0, 0)),
        pl.BlockSpec(w3.shape, lambda b: (0, 0)),
        pl.BlockSpec(a3.shape, lambda b: (0, 0)),
    ]
    return pl.pallas_call(
        _mlp_kernel,
        grid=(nb,),
        in_specs=specs,
        out_specs=pl.BlockSpec((1, NPOINT, cout), lambda b: (b, 0, 0)),
        out_shape=jax.ShapeDtypeStruct((nb, NPOINT, cout), jnp.float32),
    )(rows, q8, w80, wx8, a1, w2, a2, w3, a3)


def kernel(xyz, features, W1, b1, g1, beta1, W2, b2, g2, beta2, W3, b3, g3, beta3):
    fps_idx, new_xyz = _run_fps(xyz)
    group_idx = _run_topk(xyz, new_xyz)

    table = jnp.concatenate(
        [xyz, jnp.transpose(features, (0, 2, 1)),
         jnp.zeros((B, N, ROW - IN_CH), jnp.float32)], axis=-1,
    ).reshape(B * N, ROW)
    flat_idx = (group_idx + (jnp.arange(B, dtype=jnp.int32) * N)[:, None, None])

    q8 = jnp.pad(new_xyz, ((0, 0), (0, 0), (0, 5)))
    w80 = jnp.pad(W1, ((0, 0), (0, ROW - IN_CH)))
    wx8 = jnp.pad(W1[:, :3], ((0, 0), (0, 5)))
    params = [(w80, wx8, jnp.stack([g1, beta1, b1])),
              (W2, jnp.stack([g2, beta2, b2])),
              (W3, jnp.stack([g3, beta3, b3]))]

    H = B // 2
    outs = []
    for lo in (0, H):
        rows_h = _sc_gather(table, flat_idx[lo:lo + H].reshape(-1))
        outs.append(_run_mlp(rows_h.reshape(H, NPOINT * NSAMPLE, ROW),
                             q8[lo:lo + H], params))
    return new_xyz, jnp.concatenate(outs, axis=0)
